# baseline (speedup 1.0000x reference)
import functools

import jax
import jax.numpy as jnp
from jax.experimental import pallas as pl
from jax.experimental.pallas import tpu as pltpu

_N = 65536
_D = 512
_K = 1024
_INV_T = 2.0


def _loss_kernel(feat_ref, cent_ref, out_ref, cnorm_ref, s_ref, acc_ref, *, blk):
    i = pl.program_id(0)

    @pl.when(i == 0)
    def _prep():
        c = cent_ref[...]
        norm = jnp.sqrt(jnp.sum(c * c, axis=1, keepdims=True))
        cn = c / jnp.maximum(norm, 1e-12)
        cn16 = cn.astype(jnp.bfloat16) * jnp.bfloat16(2.0)
        cnorm_ref[...] = cn16
        g4 = jax.lax.dot_general(
            cn16, cn16, (((1,), (1,)), ((), ())),
            preferred_element_type=jnp.float32,
        )
        s_ref[...] = jnp.sum(
            jnp.exp(g4 * 0.5), axis=0, keepdims=True
        ).astype(jnp.bfloat16)
        acc_ref[...] = jnp.zeros((1, 1), jnp.float32)

    cn = cnorm_ref[...]
    s16 = s_ref[...]
    acc = jnp.zeros((1, 1), jnp.float32)
    rt = 256
    for r in range(blk // rt):
        f_r = feat_ref[pl.ds(r * rt, rt), :].astype(jnp.bfloat16)
        prod = jax.lax.dot_general(
            f_r, cn, (((1,), (1,)), ((), ())),
            preferred_element_type=jnp.float32,
        ).astype(jnp.bfloat16)
        m16 = jnp.max(prod, axis=1, keepdims=True)
        s_pick16 = jnp.min(
            jnp.where(prod == m16, s16, jnp.bfloat16(jnp.inf)),
            axis=1, keepdims=True,
        )
        m = m16.astype(jnp.float32)
        term = m - jnp.log(jnp.exp(m) + s_pick16.astype(jnp.float32))
        acc = acc + jnp.sum(term, axis=0, keepdims=True)
    acc_ref[...] += acc

    @pl.when(i == pl.num_programs(0) - 1)
    def _fin():
        out_ref[...] = -acc_ref[...] / _N


@functools.partial(jax.jit, static_argnames=("blk",))
def _run(features, centroids, blk=2048):
    out = pl.pallas_call(
        functools.partial(_loss_kernel, blk=blk),
        grid=(_N // blk,),
        in_specs=[
            pl.BlockSpec((blk, _D), lambda i: (i, 0)),
            pl.BlockSpec((_K, _D), lambda i: (0, 0)),
        ],
        out_specs=pl.BlockSpec((1, 1), lambda i: (0, 0)),
        out_shape=jax.ShapeDtypeStruct((1, 1), jnp.float32),
        scratch_shapes=[
            pltpu.VMEM((_K, _D), jnp.bfloat16),
            pltpu.VMEM((1, _K), jnp.bfloat16),
            pltpu.VMEM((1, 1), jnp.float32),
        ],
    )(features, centroids)
    return out[0, 0]


def kernel(features, centroids):
    return _run(features, centroids)


# speedup vs baseline: 1.0977x; 1.0977x over previous
import functools

import jax
import jax.numpy as jnp
from jax.experimental import pallas as pl
from jax.experimental.pallas import tpu as pltpu

_N = 65536
_D = 512
_K = 1024
_INV_T = 2.0


def _loss_kernel(feat_ref, cent_ref, out_ref, cnorm_ref, s_ref, acc_ref, *, blk):
    i = pl.program_id(0)

    @pl.when(i == 0)
    def _prep():
        c = cent_ref[...]
        norm = jnp.sqrt(jnp.sum(c * c, axis=1, keepdims=True))
        cn = c / jnp.maximum(norm, 1e-12)
        cn16 = cn.astype(jnp.bfloat16) * jnp.bfloat16(2.0)
        cnorm_ref[...] = cn16
        g4 = jax.lax.dot_general(
            cn16, cn16, (((1,), (1,)), ((), ())),
            preferred_element_type=jnp.float32,
        )
        s_ref[...] = jnp.sum(
            jnp.exp(g4 * 0.5), axis=0, keepdims=True
        ).astype(jnp.bfloat16)
        acc_ref[...] = jnp.zeros((1, 1), jnp.float32)

    cn = cnorm_ref[...]
    s16 = s_ref[...]
    acc = jnp.zeros((1, 1), jnp.float32)
    rt = 256
    for r in range(blk // rt):
        f_r = feat_ref[pl.ds(r * rt, rt), :].astype(jnp.bfloat16)
        prod = jax.lax.dot_general(
            f_r, cn, (((1,), (1,)), ((), ())),
            preferred_element_type=jnp.float32,
        ).astype(jnp.bfloat16)
        m16 = jnp.max(prod, axis=1, keepdims=True)
        s_pick16 = jnp.min(
            jnp.where(prod == m16, s16, jnp.bfloat16(jnp.inf)),
            axis=1, keepdims=True,
        )
        m = m16.astype(jnp.float32)
        term = m - jnp.log(jnp.exp(m) + s_pick16.astype(jnp.float32))
        acc = acc + jnp.sum(term, axis=0, keepdims=True)
    acc_ref[...] += acc

    @pl.when(i == pl.num_programs(0) - 1)
    def _fin():
        out_ref[...] = -acc_ref[...] / _N


@functools.partial(jax.jit, static_argnames=("blk",))
def _run(features, centroids, blk=8192):
    out = pl.pallas_call(
        functools.partial(_loss_kernel, blk=blk),
        grid=(_N // blk,),
        in_specs=[
            pl.BlockSpec((blk, _D), lambda i: (i, 0)),
            pl.BlockSpec((_K, _D), lambda i: (0, 0)),
        ],
        out_specs=pl.BlockSpec((1, 1), lambda i: (0, 0)),
        out_shape=jax.ShapeDtypeStruct((1, 1), jnp.float32),
        scratch_shapes=[
            pltpu.VMEM((_K, _D), jnp.bfloat16),
            pltpu.VMEM((1, _K), jnp.bfloat16),
            pltpu.VMEM((1, 1), jnp.float32),
        ],
    )(features, centroids)
    return out[0, 0]


def kernel(features, centroids):
    return _run(features, centroids)


# speedup vs baseline: 1.1021x; 1.0040x over previous
import functools

import jax
import jax.numpy as jnp
from jax.experimental import pallas as pl
from jax.experimental.pallas import tpu as pltpu

_N = 65536
_D = 512
_K = 1024


def _loss_kernel(feat_ref, cent_ref, out_ref, cnorm_ref, s_ref, acc_ref, *, blk):
    i = pl.program_id(0)

    @pl.when(i == 0)
    def _prep():
        c = cent_ref[...]
        norm = jnp.sqrt(jnp.sum(c * c, axis=1, keepdims=True))
        cn = c / jnp.maximum(norm, 1e-12)
        cn16 = cn.astype(jnp.bfloat16) * jnp.bfloat16(2.0)
        cnorm_ref[...] = cn16
        g4 = jax.lax.dot_general(
            cn16, cn16, (((1,), (1,)), ((), ())),
            preferred_element_type=jnp.float32,
        )
        s_ref[...] = jnp.sum(
            jnp.exp(g4 * 0.5), axis=0, keepdims=True
        ).astype(jnp.bfloat16)
        acc_ref[...] = jnp.zeros((1, 1), jnp.float32)

    cn = cnorm_ref[...]
    s16 = s_ref[...]
    acc = jnp.zeros((1, 1), jnp.float32)
    rt = 256
    for r in range(blk // rt):
        f_r = feat_ref[pl.ds(r * rt, rt), :].astype(jnp.bfloat16)
        prod = jax.lax.dot_general(
            f_r, cn, (((1,), (1,)), ((), ())),
            preferred_element_type=jnp.float32,
        ).astype(jnp.bfloat16)
        m16 = jnp.max(prod, axis=1, keepdims=True)
        s_pick16 = jnp.min(
            jnp.where(prod == m16, s16, jnp.bfloat16(jnp.inf)),
            axis=1, keepdims=True,
        )
        m = m16.astype(jnp.float32)
        term = m - jnp.log(jnp.exp(m) + s_pick16.astype(jnp.float32))
        acc = acc + jnp.sum(term, axis=0, keepdims=True)
    acc_ref[...] += acc

    @pl.when(i == pl.num_programs(0) - 1)
    def _fin():
        out_ref[...] = -acc_ref[...] / _N


@functools.partial(jax.jit, static_argnames=("blk",))
def _run(features, centroids, blk=8192):
    out = pl.pallas_call(
        functools.partial(_loss_kernel, blk=blk),
        grid=(_N // blk,),
        in_specs=[
            pl.BlockSpec((blk, _D), lambda i: (i, 0)),
            pl.BlockSpec((_K, _D), lambda i: (0, 0)),
        ],
        out_specs=pl.BlockSpec((1, 1), lambda i: (0, 0)),
        out_shape=jax.ShapeDtypeStruct((1, 1), jnp.float32),
        scratch_shapes=[
            pltpu.VMEM((_K, _D), jnp.bfloat16),
            pltpu.VMEM((1, _K), jnp.bfloat16),
            pltpu.VMEM((1, 1), jnp.float32),
        ],
    )(features, centroids)
    return out[0, 0]


def kernel(features, centroids):
    return _run(features, centroids)
